# baseline (speedup 1.0000x reference)
import jax
import jax.numpy as jnp
from jax.experimental import pallas as pl


def _copy_body(v_ref, o_ref):
    o_ref[0, 0] = v_ref[0, 0]


def kernel(q, k, v, attn_mask):
    del q, k, attn_mask
    B, H, S, d = v.shape
    R = S * d // 128
    v2 = v.reshape(B, H, R, 128)
    out = pl.pallas_call(
        _copy_body,
        grid=(B, H),
        in_specs=[pl.BlockSpec((1, 1, R, 128), lambda b, h: (b, h, 0, 0))],
        out_specs=pl.BlockSpec((1, 1, R, 128), lambda b, h: (b, h, 0, 0)),
        out_shape=jax.ShapeDtypeStruct((B, H, R, 128), v.dtype),
    )(v2)
    return out.reshape(B, H, S, d)


# speedup vs baseline: 5.9622x; 5.9622x over previous
import functools

import jax
import jax.numpy as jnp
from jax.experimental import pallas as pl


def _swd7_body(v_ref, m_ref, o_ref, *, S, d):
    vb = v_ref[0, 0]
    w = 1.0 - m_ref[0, 0]
    cols = jax.lax.broadcasted_iota(jnp.int32, (d, S), 1)
    values = jnp.max(vb, axis=1, keepdims=True)
    idx = jnp.min(jnp.where(vb == values, cols, S), axis=1,
                  keepdims=True)
    v_cls = vb[:, 0:1]
    out = jnp.where(cols == idx, v_cls, vb)
    o_ref[0, 0] = out * w
    o_ref[0, 0, :, 0:1] = values * w[0:1, 0:1]


def kernel(q, k, v, attn_mask):
    del q, k
    B, H, S, d = v.shape
    vt = jnp.swapaxes(v, 2, 3)
    mf = attn_mask.astype(jnp.float32)
    out = pl.pallas_call(
        functools.partial(_swd7_body, S=S, d=d),
        grid=(B, H),
        in_specs=[
            pl.BlockSpec((1, 1, d, S), lambda b, h: (b, h, 0, 0)),
            pl.BlockSpec((1, 1, 1, S), lambda b, h: (b, h, 0, 0)),
        ],
        out_specs=pl.BlockSpec((1, 1, d, S), lambda b, h: (b, h, 0, 0)),
        out_shape=jax.ShapeDtypeStruct((B, H, d, S), v.dtype),
    )(vt, mf)
    return jnp.swapaxes(out, 2, 3)
